# Initial kernel scaffold; baseline (speedup 1.0000x reference)
import jax
import jax.numpy as jnp
from jax.experimental import pallas as pl


def kernel(x, edge_index, W, b, gamma, beta, running_mean, running_var):
    raise NotImplementedError("write your pallas kernel here")


# speedup vs baseline: 20.4318x; 20.4318x over previous
import functools

import jax
import jax.numpy as jnp
from jax import lax
from jax.experimental import pallas as pl
from jax.experimental.pallas import tpu as pltpu
from jax.experimental.pallas import tpu_sc as plsc

N = 10000
E = 320000
D = 128

NC = 2
NS = 16
LANES = 16
NW = NC * NS
E_W = E // NW
CHUNK = 80
N_CHUNKS = E_W // CHUNK
ROWS_W = 624
ROWS_TAIL = N - NS * ROWS_W

_mesh = plsc.VectorSubcoreMesh(core_axis_name="c", subcore_axis_name="s")
_sc_params = pltpu.CompilerParams(needs_layout_passes=False)


def _zero_f32(ref, n_rows, n_cols):
    zer = jnp.zeros((LANES,), jnp.float32)

    def body(r, carry):
        for cc in range(n_cols // LANES):
            ref[r, pl.ds(cc * LANES, LANES)] = zer
        return carry

    lax.fori_loop(0, n_rows, body, 0)


@functools.partial(
    pl.kernel,
    out_type=jax.ShapeDtypeStruct((NW * N,), jnp.float32),
    mesh=_mesh,
    compiler_params=_sc_params,
    scratch_types=[
        pltpu.VMEM((E_W,), jnp.int32),
        pltpu.VMEM((N,), jnp.float32),
    ],
)
def _deg_kernel(dst_hbm, deg_hbm, dst_v, deg_v):
    wid = lax.axis_index("c") * NS + lax.axis_index("s")
    pltpu.sync_copy(dst_hbm.at[pl.ds(wid * E_W, E_W)], dst_v)

    zer = jnp.zeros((LANES,), jnp.float32)

    def zbody(i, carry):
        deg_v[pl.ds(i * LANES, LANES)] = zer
        return carry

    lax.fori_loop(0, N // LANES, zbody, 0)

    one = jnp.ones((LANES,), jnp.float32)

    def body(i, carry):
        idx = dst_v[pl.ds(i * LANES, LANES)]
        plsc.addupdate_scatter(deg_v, [idx], one)
        return carry

    lax.fori_loop(0, E_W // LANES, body, 0)
    pltpu.sync_copy(deg_v, deg_hbm.at[pl.ds(wid * N, N)])


@functools.partial(
    pl.kernel,
    out_type=jax.ShapeDtypeStruct((NC, N, D), jnp.float32),
    mesh=_mesh,
    compiler_params=_sc_params,
    scratch_types=[
        pltpu.VMEM((CHUNK,), jnp.int32),
        pltpu.VMEM((CHUNK,), jnp.int32),
        pltpu.VMEM((CHUNK, D), jnp.float32),
        pltpu.VMEM((CHUNK, D), jnp.float32),
        pltpu.VMEM_SHARED((N, D), jnp.float32),
        pltpu.SemaphoreType.DMA,
    ],
)
def _edge_kernel(src_hbm, dst_hbm, g_hbm, acc_hbm,
                 sidx_v, didx_v, rows_v, zrow_v, acc_sh, sem):
    cid = lax.axis_index("c")
    sid = lax.axis_index("s")
    wid = cid * NS + sid
    base = wid * E_W

    _zero_f32(zrow_v, CHUNK, D)
    row0 = sid * ROWS_W
    for k in range(ROWS_W // CHUNK):
        pltpu.sync_copy(zrow_v, acc_sh.at[pl.ds(row0 + k * CHUNK, CHUNK)])
    rem = ROWS_W % CHUNK
    if rem:
        pltpu.sync_copy(zrow_v.at[pl.ds(0, rem)],
                        acc_sh.at[pl.ds(row0 + (ROWS_W // CHUNK) * CHUNK, rem)])

    @pl.when(sid == 0)
    def _():
        pltpu.sync_copy(zrow_v.at[pl.ds(0, ROWS_TAIL)],
                        acc_sh.at[pl.ds(NS * ROWS_W, ROWS_TAIL)])

    plsc.subcore_barrier()

    def body(i, carry):
        off = base + i * CHUNK
        pltpu.sync_copy(src_hbm.at[pl.ds(off, CHUNK)], sidx_v)
        pltpu.sync_copy(dst_hbm.at[pl.ds(off, CHUNK)], didx_v)
        pltpu.async_copy(g_hbm.at[sidx_v], rows_v, sem).wait()
        pltpu.sync_copy(rows_v, acc_sh.at[didx_v], add=True)
        return carry

    lax.fori_loop(0, N_CHUNKS, body, 0)
    plsc.subcore_barrier()

    pltpu.sync_copy(acc_sh.at[pl.ds(row0, ROWS_W)],
                    acc_hbm.at[cid, pl.ds(row0, ROWS_W)])

    @pl.when(sid == 0)
    def _():
        pltpu.sync_copy(acc_sh.at[pl.ds(NS * ROWS_W, ROWS_TAIL)],
                        acc_hbm.at[cid, pl.ds(NS * ROWS_W, ROWS_TAIL)])


_BR = 1000


def _matmul_body(x_ref, w_ref, h_ref):
    h_ref[...] = jnp.dot(x_ref[...], w_ref[...],
                         preferred_element_type=jnp.float32,
                         precision=lax.Precision.HIGHEST)


def _matmul(x, W):
    return pl.pallas_call(
        _matmul_body,
        grid=(N // _BR,),
        in_specs=[
            pl.BlockSpec((_BR, D), lambda i: (i, 0)),
            pl.BlockSpec((D, D), lambda i: (0, 0)),
        ],
        out_specs=pl.BlockSpec((_BR, D), lambda i: (i, 0)),
        out_shape=jax.ShapeDtypeStruct((N, D), jnp.float32),
    )(x, W)


def _dis_body(degp_ref, dis_ref):
    deg = jnp.sum(degp_ref[...], axis=0) + 1.0
    dis_ref[...] = lax.rsqrt(deg)[:, None]


def _dis(deg_parts):
    return pl.pallas_call(
        _dis_body,
        out_shape=jax.ShapeDtypeStruct((N, 1), jnp.float32),
    )(deg_parts)


def _scale_body(h_ref, dis_ref, g_ref):
    g_ref[...] = h_ref[...] * dis_ref[...]


def _scale(h, dis):
    return pl.pallas_call(
        _scale_body,
        grid=(N // _BR,),
        in_specs=[
            pl.BlockSpec((_BR, D), lambda i: (i, 0)),
            pl.BlockSpec((_BR, 1), lambda i: (i, 0)),
        ],
        out_specs=pl.BlockSpec((_BR, D), lambda i: (i, 0)),
        out_shape=jax.ShapeDtypeStruct((N, D), jnp.float32),
    )(h, dis)


def _epi_body(acc_ref, g_ref, dis_ref, b_ref, sc_ref, of_ref, o_ref):
    v = (acc_ref[0] + acc_ref[1] + g_ref[...]) * dis_ref[...] + b_ref[...]
    o_ref[...] = jnp.maximum(v * sc_ref[...] + of_ref[...], 0.0)


def _epilogue(acc, g, dis, b2, scale2, offset2):
    return pl.pallas_call(
        _epi_body,
        grid=(N // _BR,),
        in_specs=[
            pl.BlockSpec((NC, _BR, D), lambda i: (0, i, 0)),
            pl.BlockSpec((_BR, D), lambda i: (i, 0)),
            pl.BlockSpec((_BR, 1), lambda i: (i, 0)),
            pl.BlockSpec((1, D), lambda i: (0, 0)),
            pl.BlockSpec((1, D), lambda i: (0, 0)),
            pl.BlockSpec((1, D), lambda i: (0, 0)),
        ],
        out_specs=pl.BlockSpec((_BR, D), lambda i: (i, 0)),
        out_shape=jax.ShapeDtypeStruct((N, D), jnp.float32),
    )(acc, g, dis, b2, scale2, offset2)


def kernel(x, edge_index, W, b, gamma, beta, running_mean, running_var):
    src = edge_index[0].astype(jnp.int32)
    dst = edge_index[1].astype(jnp.int32)

    deg_parts = _deg_kernel(dst).reshape(NW, N)
    h = _matmul(x, W)
    dis = _dis(deg_parts)
    g = _scale(h, dis)
    acc = _edge_kernel(src, dst, g)

    scale = gamma * lax.rsqrt(running_var + 1e-5)
    offset = beta - running_mean * scale
    return _epilogue(acc, g, dis, b.reshape(1, D),
                     scale.reshape(1, D), offset.reshape(1, D))


# speedup vs baseline: 39.8827x; 1.9520x over previous
import functools

import jax
import jax.numpy as jnp
from jax import lax
from jax.experimental import pallas as pl
from jax.experimental.pallas import tpu as pltpu
from jax.experimental.pallas import tpu_sc as plsc

N = 10000
E = 320000
D = 128

NC = 2
NS = 16
LANES = 16
NW = NC * NS
E_W = E // NW
CHUNK = 80
N_CHUNKS = E_W // CHUNK
ROWS_W = 624
ROWS_TAIL = N - NS * ROWS_W

_mesh = plsc.VectorSubcoreMesh(core_axis_name="c", subcore_axis_name="s")
_sc_params = pltpu.CompilerParams(needs_layout_passes=False)


def _zero_f32(ref, n_rows, n_cols):
    zer = jnp.zeros((LANES,), jnp.float32)

    def body(r, carry):
        for cc in range(n_cols // LANES):
            ref[r, pl.ds(cc * LANES, LANES)] = zer
        return carry

    lax.fori_loop(0, n_rows, body, 0)


@functools.partial(
    pl.kernel,
    out_type=jax.ShapeDtypeStruct((NW * N,), jnp.float32),
    mesh=_mesh,
    compiler_params=_sc_params,
    scratch_types=[
        pltpu.VMEM((E_W,), jnp.int32),
        pltpu.VMEM((N,), jnp.float32),
    ],
)
def _deg_kernel(dst_hbm, deg_hbm, dst_v, deg_v):
    wid = lax.axis_index("c") * NS + lax.axis_index("s")
    pltpu.sync_copy(dst_hbm.at[pl.ds(wid * E_W, E_W)], dst_v)

    zer = jnp.zeros((LANES,), jnp.float32)

    def zbody(i, carry):
        deg_v[pl.ds(i * LANES, LANES)] = zer
        return carry

    lax.fori_loop(0, N // LANES, zbody, 0)

    one = jnp.ones((LANES,), jnp.float32)

    def body(i, carry):
        idx = dst_v[pl.ds(i * LANES, LANES)]
        plsc.addupdate_scatter(deg_v, [idx], one)
        return carry

    lax.fori_loop(0, E_W // LANES, body, 0)
    pltpu.sync_copy(deg_v, deg_hbm.at[pl.ds(wid * N, N)])


NBUF = 2


@functools.partial(
    pl.kernel,
    out_type=jax.ShapeDtypeStruct((NC, N, D), jnp.float32),
    mesh=_mesh,
    compiler_params=_sc_params,
    scratch_types=[
        pltpu.VMEM((E_W,), jnp.int32),
        pltpu.VMEM((N_CHUNKS, CHUNK), jnp.int32),
        [pltpu.VMEM((CHUNK, D), jnp.float32) for _ in range(NBUF)],
        [pltpu.SemaphoreType.DMA for _ in range(NBUF)],
        pltpu.VMEM_SHARED((N, D), jnp.float32),
    ],
)
def _edge_kernel(src_hbm, dst_hbm, g_hbm, acc_hbm,
                 src_v, dst_v, rows_v, sems, acc_sh):
    cid = lax.axis_index("c")
    sid = lax.axis_index("s")
    wid = cid * NS + sid

    pltpu.sync_copy(src_hbm.at[pl.ds(wid * E_W, E_W)], src_v)
    pltpu.sync_copy(dst_hbm.at[wid], dst_v)

    zrow = rows_v[0]
    _zero_f32(zrow, CHUNK, D)
    row0 = sid * ROWS_W
    for k in range(ROWS_W // CHUNK):
        pltpu.sync_copy(zrow, acc_sh.at[pl.ds(row0 + k * CHUNK, CHUNK)])
    rem = ROWS_W % CHUNK
    if rem:
        pltpu.sync_copy(zrow.at[pl.ds(0, rem)],
                        acc_sh.at[pl.ds(row0 + (ROWS_W // CHUNK) * CHUNK, rem)])

    @pl.when(sid == 0)
    def _():
        pltpu.sync_copy(zrow.at[pl.ds(0, ROWS_TAIL)],
                        acc_sh.at[pl.ds(NS * ROWS_W, ROWS_TAIL)])

    plsc.subcore_barrier()

    def start_gather(i, b):
        pltpu.async_copy(g_hbm.at[src_v.at[pl.ds(i * CHUNK, CHUNK)]],
                         rows_v[b], sems[b])

    def wait_gather(i, b):
        pltpu.make_async_copy(g_hbm.at[src_v.at[pl.ds(i * CHUNK, CHUNK)]],
                              rows_v[b], sems[b]).wait()

    for b in range(NBUF):
        start_gather(b, b)

    def body(jblk, carry):
        j0 = jblk * NBUF
        for b in range(NBUF):
            i = j0 + b
            wait_gather(i, b)
            pltpu.sync_copy(rows_v[b], acc_sh.at[dst_v.at[i]], add=True)

            @pl.when(i + NBUF < N_CHUNKS)
            def _():
                start_gather(i + NBUF, b)

        return carry

    n_full = N_CHUNKS // NBUF
    lax.fori_loop(0, n_full, body, 0)
    for i in range(n_full * NBUF, N_CHUNKS):
        b = i % NBUF
        wait_gather(i, b)
        pltpu.sync_copy(rows_v[b], acc_sh.at[dst_v.at[i]], add=True)
    plsc.subcore_barrier()

    pltpu.sync_copy(acc_sh.at[pl.ds(row0, ROWS_W)],
                    acc_hbm.at[cid, pl.ds(row0, ROWS_W)])

    @pl.when(sid == 0)
    def _():
        pltpu.sync_copy(acc_sh.at[pl.ds(NS * ROWS_W, ROWS_TAIL)],
                        acc_hbm.at[cid, pl.ds(NS * ROWS_W, ROWS_TAIL)])


_BR = 1000


def _matmul_body(x_ref, w_ref, h_ref):
    h_ref[...] = jnp.dot(x_ref[...], w_ref[...],
                         preferred_element_type=jnp.float32,
                         precision=lax.Precision.HIGHEST)


def _matmul(x, W):
    return pl.pallas_call(
        _matmul_body,
        grid=(N // _BR,),
        in_specs=[
            pl.BlockSpec((_BR, D), lambda i: (i, 0)),
            pl.BlockSpec((D, D), lambda i: (0, 0)),
        ],
        out_specs=pl.BlockSpec((_BR, D), lambda i: (i, 0)),
        out_shape=jax.ShapeDtypeStruct((N, D), jnp.float32),
    )(x, W)


def _dis_body(degp_ref, dis_ref):
    deg = jnp.sum(degp_ref[...], axis=0) + 1.0
    dis_ref[...] = lax.rsqrt(deg)[:, None]


def _dis(deg_parts):
    return pl.pallas_call(
        _dis_body,
        out_shape=jax.ShapeDtypeStruct((N, 1), jnp.float32),
    )(deg_parts)


def _scale_body(h_ref, dis_ref, g_ref):
    g_ref[...] = h_ref[...] * dis_ref[...]


def _scale(h, dis):
    return pl.pallas_call(
        _scale_body,
        grid=(N // _BR,),
        in_specs=[
            pl.BlockSpec((_BR, D), lambda i: (i, 0)),
            pl.BlockSpec((_BR, 1), lambda i: (i, 0)),
        ],
        out_specs=pl.BlockSpec((_BR, D), lambda i: (i, 0)),
        out_shape=jax.ShapeDtypeStruct((N, D), jnp.float32),
    )(h, dis)


def _epi_body(acc_ref, g_ref, dis_ref, b_ref, sc_ref, of_ref, o_ref):
    v = (acc_ref[0] + acc_ref[1] + g_ref[...]) * dis_ref[...] + b_ref[...]
    o_ref[...] = jnp.maximum(v * sc_ref[...] + of_ref[...], 0.0)


def _epilogue(acc, g, dis, b2, scale2, offset2):
    return pl.pallas_call(
        _epi_body,
        grid=(N // _BR,),
        in_specs=[
            pl.BlockSpec((NC, _BR, D), lambda i: (0, i, 0)),
            pl.BlockSpec((_BR, D), lambda i: (i, 0)),
            pl.BlockSpec((_BR, 1), lambda i: (i, 0)),
            pl.BlockSpec((1, D), lambda i: (0, 0)),
            pl.BlockSpec((1, D), lambda i: (0, 0)),
            pl.BlockSpec((1, D), lambda i: (0, 0)),
        ],
        out_specs=pl.BlockSpec((_BR, D), lambda i: (i, 0)),
        out_shape=jax.ShapeDtypeStruct((N, D), jnp.float32),
    )(acc, g, dis, b2, scale2, offset2)


def kernel(x, edge_index, W, b, gamma, beta, running_mean, running_var):
    src = edge_index[0].astype(jnp.int32)
    dst = edge_index[1].astype(jnp.int32)

    deg_parts = _deg_kernel(dst).reshape(NW, N)
    h = _matmul(x, W)
    dis = _dis(deg_parts)
    g = _scale(h, dis)
    acc = _edge_kernel(src, dst.reshape(NW, N_CHUNKS, CHUNK), g)

    scale = gamma * lax.rsqrt(running_var + 1e-5)
    offset = beta - running_mean * scale
    return _epilogue(acc, g, dis, b.reshape(1, D),
                     scale.reshape(1, D), offset.reshape(1, D))
